# baseline (speedup 1.0000x reference)
import functools

import jax
import jax.numpy as jnp
from jax import lax
from jax.experimental import pallas as pl
from jax.experimental.pallas import tpu as pltpu
from jax.experimental.pallas import tpu_sc as plsc

B = 16384
NF = 26
DIM = 16
VOCAB = 100000

NC = 2
NS = 16
NW = NC * NS
B_PER_W = B // NW
CS = 128
NCHUNK = B_PER_W // CS
CF = CS * NF
JB = 4
UPB = B * 8
NUNIT = JB * UPB


@functools.lru_cache(maxsize=None)
def _sc_gather_build():
    mesh = plsc.VectorSubcoreMesh(core_axis_name="c", subcore_axis_name="s",
                                  num_cores=NC, num_subcores=NS)

    @functools.partial(
        pl.kernel,
        mesh=mesh,
        out_type=jax.ShapeDtypeStruct((NUNIT, DIM), jnp.float32),
        scratch_types=[
            pltpu.VMEM((NF, B_PER_W), jnp.int32),
            pltpu.VMEM((CF,), jnp.int32),
            pltpu.VMEM((CF,), jnp.int32),
            pltpu.VMEM((NF, CS), jnp.int32),
            pltpu.VMEM((NF, CS), jnp.int32),
            pltpu.VMEM((NF, CS), jnp.int32),
            pltpu.VMEM((1, CS), jnp.int32),
            pltpu.VMEM((NF, CS, DIM), jnp.float32),
            pltpu.VMEM((NF, CS), jnp.float32),
            pltpu.VMEM((CS, DIM), jnp.float32),
            pltpu.SemaphoreType.DMA,
            pltpu.SemaphoreType.DMA,
            pltpu.SemaphoreType.DMA,
        ],
        compiler_params=pltpu.CompilerParams(use_tc_tiling_on_sc=False,
                                             needs_layout_passes=False),
    )
    def sc_gather(*refs):
        c_hbm = refs[:NF]
        (fpat_hbm, bpat_hbm, emb_hbm, fm_hbm, emb_out,
         cbuf, fpat, bpat, eidx, uidx, fidx, fuidx, rows, fmv, fmrows,
         gsem, fsem, ssem) = refs[NF:]
        wid = lax.axis_index("s") * NC + lax.axis_index("c")
        base_w = wid * B_PER_W

        for f in range(NF):
            pltpu.async_copy(c_hbm[f].at[pl.ds(base_w, B_PER_W)],
                             cbuf.at[f], gsem)
        pltpu.async_copy(fpat_hbm, fpat, gsem)
        pltpu.async_copy(bpat_hbm, bpat, gsem)
        for f in range(NF):
            pltpu.make_async_copy(c_hbm[f].at[pl.ds(base_w, B_PER_W)],
                                  cbuf.at[f], gsem).wait()
        pltpu.make_async_copy(fpat_hbm, fpat, gsem).wait()
        pltpu.make_async_copy(bpat_hbm, bpat, gsem).wait()

        def zgrp(s, carry):
            fmrows[s, :] = jnp.zeros((DIM,), jnp.float32)
            return carry

        lax.fori_loop(0, CS, zgrp, 0)

        def chunk_body(c, carry):
            base_b = base_w + c * CS

            def grp(g, carry):
                j = g // (CS // 16)
                k = g % (CS // 16)
                fv = fpat[pl.ds(g * 16, 16)]
                bv = bpat[pl.ds(g * 16, 16)]
                v = plsc.load_gather(cbuf, [fv, bv + c * CS])
                eidx[j, pl.ds(k * 16, 16)] = v + fv * VP
                uidx[j, pl.ds(k * 16, 16)] = (
                    lax.shift_right_logical(fv, 3) * UPB
                    + (base_b + bv) * 8 + lax.bitwise_and(fv, 7))
                return carry

            lax.fori_loop(0, CF // 16, grp, 0)

            def fgrp(g, carry):
                j = g // (CS // 16)
                k = g % (CS // 16)
                fidx[j, pl.ds(k * 16, 16)] = \
                    cbuf[j, pl.ds(c * CS + k * 16, 16)] + VOCAB
                return carry

            lax.fori_loop(0, NF * (CS // 16), fgrp, 0)

            def kgrp(k, carry):
                lane = lax.broadcasted_iota(jnp.int32, (16,), 0)
                fuidx[0, pl.ds(k * 16, 16)] = \
                    3 * UPB + (base_b + k * 16 + lane) * 8 + 2
                return carry

            lax.fori_loop(0, CS // 16, kgrp, 0)

            def fire(j, carry):
                pltpu.async_copy(emb_hbm.at[eidx.at[j]], rows.at[j], gsem)
                pltpu.async_copy(fm_hbm.at[fidx.at[j]], fmv.at[j], fsem)
                return carry

            lax.fori_loop(0, NF, fire, 0)

            def drain(j, carry):
                pltpu.make_async_copy(
                    emb_hbm.at[eidx.at[j]], rows.at[j], gsem).wait()
                pltpu.make_async_copy(
                    fm_hbm.at[fidx.at[j]], fmv.at[j], fsem).wait()
                return carry

            lax.fori_loop(0, NF, drain, 0)

            def fsum_grp(k, carry):
                acc = fmv[0, pl.ds(k * 16, 16)]

                def facc(f, a):
                    return a + fmv[f, pl.ds(k * 16, 16)]

                acc = lax.fori_loop(1, NF, facc, acc)
                lane = lax.broadcasted_iota(jnp.int32, (16,), 0)
                plsc.store_scatter(fmrows, [k * 16 + lane, lane * 0], acc)
                return carry

            lax.fori_loop(0, CS // 16, fsum_grp, 0)

            def sfire(j, carry):
                pltpu.async_copy(rows.at[j], emb_out.at[uidx.at[j]], ssem)
                return carry

            lax.fori_loop(0, NF, sfire, 0)
            pltpu.async_copy(fmrows, emb_out.at[fuidx.at[0]], ssem)

            def sdrain(j, carry):
                pltpu.make_async_copy(
                    rows.at[j], emb_out.at[uidx.at[j]], ssem).wait()
                return carry

            lax.fori_loop(0, NF, sdrain, 0)
            pltpu.make_async_copy(fmrows, emb_out.at[fuidx.at[0]], ssem).wait()
            return carry

        lax.fori_loop(0, NCHUNK, chunk_body, 0)

    return sc_gather


VBLK = 2048
NVB = (VOCAB + VBLK - 1) // VBLK
VP = NVB * VBLK


def _retile_body(in_ref, o_ref):
    x = in_ref[0]
    t = jnp.transpose(x)
    t3 = t.reshape(VBLK // 8, 8, DIM)
    o_ref[0] = jnp.concatenate([t3[:, k, :] for k in range(8)], axis=1)


def _retile(embT):
    return pl.pallas_call(
        _retile_body,
        grid=(NF, NVB),
        in_specs=[pl.BlockSpec((1, DIM, VBLK), lambda f, j: (f, 0, j))],
        out_specs=pl.BlockSpec((1, VBLK // 8, 128), lambda f, j: (f, j, 0)),
        out_shape=jax.ShapeDtypeStruct(
            (NF, VP * DIM // 128, 128), jnp.float32),
    )(embT)


def _tc_body(x0_ref, x1_ref, x2_ref, x3_ref, w1_ref, b1_ref, w2_ref, b2_ref,
             w3_ref, b3_ref, wd_ref, bd_ref, o_ref):
    lane = lax.broadcasted_iota(jnp.int32, (1, 128), 1)
    x3 = x3_ref[...]
    x3a = jnp.where(lane < 32, x3, 0.0)
    x3f = jnp.where(lane < 48, x3, 0.0)
    xb = [x0_ref[...], x1_ref[...], x2_ref[...], x3a]

    def blk_dot(xs, w_ref):
        acc = jnp.dot(xs[0], w_ref[pl.ds(0, 128), :],
                      preferred_element_type=jnp.float32)
        for jb in range(1, JB):
            acc = acc + jnp.dot(xs[jb], w_ref[pl.ds(jb * 128, 128), :],
                                preferred_element_type=jnp.float32)
        return acc

    h = jnp.maximum(blk_dot(xb, w1_ref) + b1_ref[...], 0.0)
    h = jnp.maximum(
        jnp.dot(h, w2_ref[...], preferred_element_type=jnp.float32)
        + b2_ref[...], 0.0)
    h = jnp.maximum(
        jnp.dot(h, w3_ref[...], preferred_element_type=jnp.float32)
        + b3_ref[...], 0.0)
    deep = jnp.dot(h, wd_ref[...], preferred_element_type=jnp.float32) \
        + bd_ref[...]
    rows_i = lax.broadcasted_iota(jnp.int32, (128, DIM), 0)
    cols_i = lax.broadcasted_iota(jnp.int32, (128, DIM), 1)
    S = (rows_i % DIM == cols_i).astype(jnp.float32)
    s1 = jnp.dot(xb[0], S, preferred_element_type=jnp.float32)
    s2 = jnp.dot(xb[0] * xb[0], S, preferred_element_type=jnp.float32)
    for jb in range(1, JB):
        s1 = s1 + jnp.dot(xb[jb], S, preferred_element_type=jnp.float32)
        s2 = s2 + jnp.dot(xb[jb] * xb[jb], S,
                          preferred_element_type=jnp.float32)
    second = 0.5 * jnp.sum(s1 * s1 - s2, axis=1, keepdims=True)
    sel = (lax.broadcasted_iota(jnp.int32, (128, 1), 0) == 32) \
        .astype(jnp.float32)
    first = jnp.dot(x3f, sel, preferred_element_type=jnp.float32)
    z = first + second + deep
    o_ref[...] = 1.0 / (1.0 + jnp.exp(-z))


def _tc_mlp(X4, W1p, b1, W2, b2, W3, b3, Wd, bd):
    TB = 512
    grid = (B // TB,)
    nblk = B // TB

    def xspec(jb):
        return pl.BlockSpec((TB, 128), lambda i, jb=jb: (jb * nblk + i, 0))

    return pl.pallas_call(
        _tc_body,
        grid=grid,
        in_specs=[
            xspec(0), xspec(1), xspec(2), xspec(3),
            pl.BlockSpec(W1p.shape, lambda i: (0, 0)),
            pl.BlockSpec(b1.shape, lambda i: (0, 0)),
            pl.BlockSpec(W2.shape, lambda i: (0, 0)),
            pl.BlockSpec(b2.shape, lambda i: (0, 0)),
            pl.BlockSpec(W3.shape, lambda i: (0, 0)),
            pl.BlockSpec(b3.shape, lambda i: (0, 0)),
            pl.BlockSpec(Wd.shape, lambda i: (0, 0)),
            pl.BlockSpec(bd.shape, lambda i: (0, 0)),
        ],
        out_specs=pl.BlockSpec((TB, 1), lambda i: (i, 0)),
        out_shape=jax.ShapeDtypeStruct((B, 1), jnp.float32),
    )(X4, X4, X4, X4, W1p, b1, W2, b2, W3, b3, Wd, bd)


def kernel(C1, C2, C3, C4, C5, C6, C7, C8, C9, C10, C11, C12, C13, C14, C15,
           C16, C17, C18, C19, C20, C21, C22, C23, C24, C25, C26, emb_tables,
           fm_w, W1, b1, W2, b2, W3, b3, Wd, bd):
    fields = [C1, C2, C3, C4, C5, C6, C7, C8, C9, C10, C11, C12, C13, C14,
              C15, C16, C17, C18, C19, C20, C21, C22, C23, C24, C25, C26]
    embT = jnp.transpose(emb_tables, (0, 2, 1))
    emb_flat = _retile(embT).reshape(NF * VP, DIM)
    fm_flat = fm_w.reshape(-1)
    p = jnp.arange(CF, dtype=jnp.int32)
    emb_g = _sc_gather_build()(*fields, p % NF, p // NF, emb_flat, fm_flat)
    X4 = emb_g.reshape(JB * B, 128)
    W1p = jnp.pad(W1, ((0, JB * 128 - NF * DIM), (0, 0)))
    return _tc_mlp(X4, W1p, b1.reshape(1, -1), W2, b2.reshape(1, -1),
                   W3, b3.reshape(1, -1), Wd, bd.reshape(1, 1))


# speedup vs baseline: 1.8473x; 1.8473x over previous
import functools

import jax
import jax.numpy as jnp
from jax import lax
from jax.experimental import pallas as pl
from jax.experimental.pallas import tpu as pltpu
from jax.experimental.pallas import tpu_sc as plsc

B = 16384
NF = 26
DIM = 16
VOCAB = 100000

NC = 2
NS = 16
NW = NC * NS
B_PER_W = B // NW
CS = 128
NCHUNK = B_PER_W // CS
NBT = B // CS

VBLK = 4096
NVB = 25
VP = NVB * VBLK
NFP = 32

NJ = NF * DIM
RPT = NJ + 8


def _slice_body(*refs):
    x = refs[0][...]
    pad = jnp.zeros((NFP - NF, VBLK), jnp.float32)
    for d in range(DIM):
        refs[1 + d][...] = jnp.concatenate([x[:, d, :], pad], axis=0)


def _slice_tables(embT):
    return pl.pallas_call(
        _slice_body,
        grid=(NVB,),
        in_specs=[pl.BlockSpec((NF, DIM, VBLK), lambda j: (0, 0, j))],
        out_specs=[pl.BlockSpec((NFP, VBLK), lambda j: (0, j))
                   for _ in range(DIM)],
        out_shape=[jax.ShapeDtypeStruct((NFP, VP), jnp.float32)
                   for _ in range(DIM)],
    )(embT)


@functools.lru_cache(maxsize=None)
def _sc_gather_build():
    mesh = plsc.VectorSubcoreMesh(core_axis_name="c", subcore_axis_name="s",
                                  num_cores=NC, num_subcores=NS)

    @functools.partial(
        pl.kernel,
        mesh=mesh,
        out_type=jax.ShapeDtypeStruct((NBT * RPT, CS), jnp.float32),
        scratch_types=[
            pltpu.VMEM((NF, B_PER_W), jnp.int32),
            pltpu.VMEM((NF, CS), jnp.int32),
            pltpu.VMEM((NF, CS), jnp.int32),
            pltpu.VMEM((RPT, CS), jnp.float32),
            pltpu.VMEM((NF, CS), jnp.float32),
            pltpu.SemaphoreType.DMA,
            pltpu.SemaphoreType.DMA,
        ],
        compiler_params=pltpu.CompilerParams(use_tc_tiling_on_sc=False,
                                             needs_layout_passes=False),
    )
    def sc_gather(*refs):
        c_hbm = refs[:NF]
        d_tab = refs[NF:NF + DIM]
        fm_hbm = refs[NF + DIM]
        xt_out = refs[NF + DIM + 1]
        (cbuf, idx2, fidx, trbuf, fmv, gsem, fsem) = refs[NF + DIM + 2:]
        wid = lax.axis_index("s") * NC + lax.axis_index("c")
        base_w = wid * B_PER_W

        for f in range(NF):
            pltpu.async_copy(c_hbm[f].at[pl.ds(base_w, B_PER_W)],
                             cbuf.at[f], gsem)
        for f in range(NF):
            pltpu.make_async_copy(c_hbm[f].at[pl.ds(base_w, B_PER_W)],
                                  cbuf.at[f], gsem).wait()

        def chunk_body(c, carry):
            bt = wid * NCHUNK + c

            def igrp(g, carry):
                f = g // (CS // 16)
                k = g % (CS // 16)
                v = cbuf[f, pl.ds(c * CS + k * 16, 16)]
                idx2[f, pl.ds(k * 16, 16)] = v + f * VP
                fidx[f, pl.ds(k * 16, 16)] = v + VOCAB
                return carry

            lax.fori_loop(0, NF * (CS // 16), igrp, 0)

            def fire(f, carry):
                for d in range(DIM):
                    pltpu.async_copy(d_tab[d].at[idx2.at[f]],
                                     trbuf.at[f * DIM + d], gsem)
                pltpu.async_copy(fm_hbm.at[fidx.at[f]], fmv.at[f], fsem)
                return carry

            lax.fori_loop(0, NF, fire, 0)

            def drain(f, carry):
                for d in range(DIM):
                    pltpu.make_async_copy(d_tab[d].at[idx2.at[f]],
                                          trbuf.at[f * DIM + d], gsem).wait()
                pltpu.make_async_copy(fm_hbm.at[fidx.at[f]],
                                      fmv.at[f], fsem).wait()
                return carry

            lax.fori_loop(0, NF, drain, 0)

            def fsum_grp(k, carry):
                acc = fmv[0, pl.ds(k * 16, 16)]

                def facc(f, a):
                    return a + fmv[f, pl.ds(k * 16, 16)]

                trbuf[NJ, pl.ds(k * 16, 16)] = lax.fori_loop(1, NF, facc, acc)
                return carry

            lax.fori_loop(0, CS // 16, fsum_grp, 0)

            pltpu.sync_copy(trbuf, xt_out.at[pl.ds(bt * RPT, RPT)])
            return carry

        lax.fori_loop(0, NCHUNK, chunk_body, 0)

    return sc_gather


def _tc_body(x_ref, w1_ref, b1_ref, w2_ref, b2_ref, w3_ref, b3_ref,
             wd_ref, bd_ref, o_ref):
    di = lax.broadcasted_iota(jnp.int32, (DIM, NJ), 0)
    ji = lax.broadcasted_iota(jnp.int32, (DIM, NJ), 1)
    ST = (ji % DIM == di).astype(jnp.float32)
    zs = []
    for g in range(8):
        xe = x_ref[pl.ds(g * RPT, NJ), :]
        first = x_ref[pl.ds(g * RPT + NJ, 8), :][0:1, :]
        h = jnp.maximum(
            jnp.dot(w1_ref[...], xe, preferred_element_type=jnp.float32)
            + b1_ref[...], 0.0)
        h = jnp.maximum(
            jnp.dot(w2_ref[...], h, preferred_element_type=jnp.float32)
            + b2_ref[...], 0.0)
        h = jnp.maximum(
            jnp.dot(w3_ref[...], h, preferred_element_type=jnp.float32)
            + b3_ref[...], 0.0)
        deep = jnp.dot(wd_ref[...], h, preferred_element_type=jnp.float32) \
            + bd_ref[...]
        s1 = jnp.dot(ST, xe, preferred_element_type=jnp.float32)
        s2 = jnp.dot(ST, xe * xe, preferred_element_type=jnp.float32)
        second = 0.5 * jnp.sum(s1 * s1 - s2, axis=0, keepdims=True)
        z = first + second + deep
        zs.append(1.0 / (1.0 + jnp.exp(-z)))
    o_ref[...] = jnp.concatenate(zs, axis=0)


def _tc_mlp(XT, W1T, b1, W2T, b2, W3T, b3, WdT, bd):
    grid = (NBT // 8,)
    return pl.pallas_call(
        _tc_body,
        grid=grid,
        in_specs=[
            pl.BlockSpec((8 * RPT, CS), lambda i: (i, 0)),
            pl.BlockSpec(W1T.shape, lambda i: (0, 0)),
            pl.BlockSpec(b1.shape, lambda i: (0, 0)),
            pl.BlockSpec(W2T.shape, lambda i: (0, 0)),
            pl.BlockSpec(b2.shape, lambda i: (0, 0)),
            pl.BlockSpec(W3T.shape, lambda i: (0, 0)),
            pl.BlockSpec(b3.shape, lambda i: (0, 0)),
            pl.BlockSpec(WdT.shape, lambda i: (0, 0)),
            pl.BlockSpec(bd.shape, lambda i: (0, 0)),
        ],
        out_specs=pl.BlockSpec((8, CS), lambda i: (i, 0)),
        out_shape=jax.ShapeDtypeStruct((NBT, CS), jnp.float32),
    )(XT, W1T, b1, W2T, b2, W3T, b3, WdT, bd)


def kernel(C1, C2, C3, C4, C5, C6, C7, C8, C9, C10, C11, C12, C13, C14, C15,
           C16, C17, C18, C19, C20, C21, C22, C23, C24, C25, C26, emb_tables,
           fm_w, W1, b1, W2, b2, W3, b3, Wd, bd):
    fields = [C1, C2, C3, C4, C5, C6, C7, C8, C9, C10, C11, C12, C13, C14,
              C15, C16, C17, C18, C19, C20, C21, C22, C23, C24, C25, C26]
    embT = jnp.transpose(emb_tables, (0, 2, 1))
    d_tabs = [t.reshape(-1) for t in _slice_tables(embT)]
    fm_flat = fm_w.reshape(-1)
    XT = _sc_gather_build()(*fields, *d_tabs, fm_flat)
    out = _tc_mlp(XT, W1.T, b1.reshape(-1, 1), W2.T, b2.reshape(-1, 1),
                  W3.T, b3.reshape(-1, 1), Wd.T, bd.reshape(1, 1))
    return out.reshape(B, 1)


# speedup vs baseline: 2.2643x; 1.2258x over previous
import functools

import jax
import jax.numpy as jnp
from jax import lax
from jax.experimental import pallas as pl
from jax.experimental.pallas import tpu as pltpu
from jax.experimental.pallas import tpu_sc as plsc

B = 16384
NF = 26
DIM = 16
VOCAB = 100000

NC = 2
NS = 16
NW = NC * NS
B_PER_W = B // NW
CS = 128
NCHUNK = B_PER_W // CS
NBT = B // CS

VBLK = 2048
VSH = 11
NVB = 49
VP = NVB * VBLK
NFP = 32

NJ = NF * DIM
RPT = NJ + 8


def _slice_body(*refs):
    x = refs[0][...]
    pad = jnp.zeros(((NFP - NF) * VBLK // 128, 128), jnp.float32)
    for d in range(DIM):
        xd = x[:, d, :].reshape(NF * VBLK // 128, 128)
        refs[1 + d][...] = jnp.concatenate([xd, pad], axis=0)


def _slice_tables(embT):
    return pl.pallas_call(
        _slice_body,
        grid=(NVB,),
        in_specs=[pl.BlockSpec((NF, DIM, VBLK), lambda j: (0, 0, j))],
        out_specs=[pl.BlockSpec((NFP * VBLK // 128, 128), lambda j: (j, 0))
                   for _ in range(DIM)],
        out_shape=[jax.ShapeDtypeStruct((NVB * NFP * VBLK // 128, 128),
                                        jnp.float32)
                   for _ in range(DIM)],
    )(embT)


@functools.lru_cache(maxsize=None)
def _sc_gather_build():
    mesh = plsc.VectorSubcoreMesh(core_axis_name="c", subcore_axis_name="s",
                                  num_cores=NC, num_subcores=NS)

    @functools.partial(
        pl.kernel,
        mesh=mesh,
        out_type=jax.ShapeDtypeStruct((NBT * RPT, CS), jnp.float32),
        scratch_types=[
            pltpu.VMEM((NF, B_PER_W), jnp.int32),
            pltpu.VMEM((NF, CS), jnp.int32),
            pltpu.VMEM((NF, CS), jnp.int32),
            pltpu.VMEM((RPT, CS), jnp.float32),
            pltpu.VMEM((NF, CS), jnp.float32),
            pltpu.SemaphoreType.DMA,
            pltpu.SemaphoreType.DMA,
        ],
        compiler_params=pltpu.CompilerParams(use_tc_tiling_on_sc=False,
                                             needs_layout_passes=False),
    )
    def sc_gather(*refs):
        c_hbm = refs[:NF]
        d_tab = refs[NF:NF + DIM]
        fm_hbm = refs[NF + DIM]
        xt_out = refs[NF + DIM + 1]
        (cbuf, idx2, fidx, trbuf, fmv, gsem, fsem) = refs[NF + DIM + 2:]
        wid = lax.axis_index("s") * NC + lax.axis_index("c")
        base_w = wid * B_PER_W

        for f in range(NF):
            pltpu.async_copy(c_hbm[f].at[pl.ds(base_w, B_PER_W)],
                             cbuf.at[f], gsem)
        for f in range(NF):
            pltpu.make_async_copy(c_hbm[f].at[pl.ds(base_w, B_PER_W)],
                                  cbuf.at[f], gsem).wait()

        def chunk_body(c, carry):
            bt = wid * NCHUNK + c

            def igrp(g, carry):
                f = g // (CS // 16)
                k = g % (CS // 16)
                v = cbuf[f, pl.ds(c * CS + k * 16, 16)]
                idx2[f, pl.ds(k * 16, 16)] = (
                    (lax.shift_right_logical(v, VSH) * NFP + f) * VBLK
                    + lax.bitwise_and(v, VBLK - 1))
                fidx[f, pl.ds(k * 16, 16)] = v + VOCAB
                return carry

            lax.fori_loop(0, NF * (CS // 16), igrp, 0)

            def fire(f, carry):
                for d in range(DIM):
                    pltpu.async_copy(d_tab[d].at[idx2.at[f]],
                                     trbuf.at[f * DIM + d], gsem)
                pltpu.async_copy(fm_hbm.at[fidx.at[f]], fmv.at[f], fsem)
                return carry

            lax.fori_loop(0, NF, fire, 0)

            def drain(f, carry):
                for d in range(DIM):
                    pltpu.make_async_copy(d_tab[d].at[idx2.at[f]],
                                          trbuf.at[f * DIM + d], gsem).wait()
                pltpu.make_async_copy(fm_hbm.at[fidx.at[f]],
                                      fmv.at[f], fsem).wait()
                return carry

            lax.fori_loop(0, NF, drain, 0)

            def fsum_grp(k, carry):
                acc = fmv[0, pl.ds(k * 16, 16)]

                def facc(f, a):
                    return a + fmv[f, pl.ds(k * 16, 16)]

                trbuf[NJ, pl.ds(k * 16, 16)] = lax.fori_loop(1, NF, facc, acc)
                return carry

            lax.fori_loop(0, CS // 16, fsum_grp, 0)

            pltpu.sync_copy(trbuf, xt_out.at[pl.ds(bt * RPT, RPT)])
            return carry

        lax.fori_loop(0, NCHUNK, chunk_body, 0)

    return sc_gather


def _tc_body(x_ref, w1_ref, b1_ref, w2_ref, b2_ref, w3_ref, b3_ref,
             wd_ref, bd_ref, o_ref):
    di = lax.broadcasted_iota(jnp.int32, (DIM, NJ), 0)
    ji = lax.broadcasted_iota(jnp.int32, (DIM, NJ), 1)
    ST = (ji % DIM == di).astype(jnp.float32)
    zs = []
    for g in range(8):
        xe = x_ref[pl.ds(g * RPT, NJ), :]
        first = x_ref[pl.ds(g * RPT + NJ, 8), :][0:1, :]
        h = jnp.maximum(
            jnp.dot(w1_ref[...], xe, preferred_element_type=jnp.float32)
            + b1_ref[...], 0.0)
        h = jnp.maximum(
            jnp.dot(w2_ref[...], h, preferred_element_type=jnp.float32)
            + b2_ref[...], 0.0)
        h = jnp.maximum(
            jnp.dot(w3_ref[...], h, preferred_element_type=jnp.float32)
            + b3_ref[...], 0.0)
        deep = jnp.dot(wd_ref[...], h, preferred_element_type=jnp.float32) \
            + bd_ref[...]
        s1 = jnp.dot(ST, xe, preferred_element_type=jnp.float32)
        s2 = jnp.dot(ST, xe * xe, preferred_element_type=jnp.float32)
        second = 0.5 * jnp.sum(s1 * s1 - s2, axis=0, keepdims=True)
        z = first + second + deep
        zs.append(1.0 / (1.0 + jnp.exp(-z)))
    o_ref[...] = jnp.concatenate(zs, axis=0)


def _tc_mlp(XT, W1T, b1, W2T, b2, W3T, b3, WdT, bd):
    grid = (NBT // 8,)
    return pl.pallas_call(
        _tc_body,
        grid=grid,
        in_specs=[
            pl.BlockSpec((8 * RPT, CS), lambda i: (i, 0)),
            pl.BlockSpec(W1T.shape, lambda i: (0, 0)),
            pl.BlockSpec(b1.shape, lambda i: (0, 0)),
            pl.BlockSpec(W2T.shape, lambda i: (0, 0)),
            pl.BlockSpec(b2.shape, lambda i: (0, 0)),
            pl.BlockSpec(W3T.shape, lambda i: (0, 0)),
            pl.BlockSpec(b3.shape, lambda i: (0, 0)),
            pl.BlockSpec(WdT.shape, lambda i: (0, 0)),
            pl.BlockSpec(bd.shape, lambda i: (0, 0)),
        ],
        out_specs=pl.BlockSpec((8, CS), lambda i: (i, 0)),
        out_shape=jax.ShapeDtypeStruct((NBT, CS), jnp.float32),
    )(XT, W1T, b1, W2T, b2, W3T, b3, WdT, bd)


def kernel(C1, C2, C3, C4, C5, C6, C7, C8, C9, C10, C11, C12, C13, C14, C15,
           C16, C17, C18, C19, C20, C21, C22, C23, C24, C25, C26, emb_tables,
           fm_w, W1, b1, W2, b2, W3, b3, Wd, bd):
    fields = [C1, C2, C3, C4, C5, C6, C7, C8, C9, C10, C11, C12, C13, C14,
              C15, C16, C17, C18, C19, C20, C21, C22, C23, C24, C25, C26]
    embT = jnp.transpose(emb_tables, (0, 2, 1))
    d_tabs = [t.reshape(-1) for t in _slice_tables(embT)]
    fm_flat = fm_w.reshape(-1)
    XT = _sc_gather_build()(*fields, *d_tabs, fm_flat)
    out = _tc_mlp(XT, W1.T, b1.reshape(-1, 1), W2.T, b2.reshape(-1, 1),
                  W3.T, b3.reshape(-1, 1), Wd.T, bd.reshape(1, 1))
    return out.reshape(B, 1)


# speedup vs baseline: 2.3076x; 1.0191x over previous
import functools

import jax
import jax.numpy as jnp
from jax import lax
from jax.experimental import pallas as pl
from jax.experimental.pallas import tpu as pltpu
from jax.experimental.pallas import tpu_sc as plsc

B = 16384
NF = 26
DIM = 16
VOCAB = 100000

NC = 2
NS = 16
NW = NC * NS
B_PER_W = B // NW
CS = 128
NCHUNK = B_PER_W // CS
NBT = B // CS

VBLK = 2048
VSH = 11
NVB = 49
VP = NVB * VBLK
NFP = 26

NJ = NF * DIM
RPT = NJ + 8


def _slice_body(*refs):
    x = refs[0][...]
    for d in range(DIM):
        refs[1 + d][...] = x[:, d, :].reshape(NF * VBLK // 128, 128)


def _slice_tables(embT):
    return pl.pallas_call(
        _slice_body,
        grid=(NVB,),
        in_specs=[pl.BlockSpec((NF, DIM, VBLK), lambda j: (0, 0, j))],
        out_specs=[pl.BlockSpec((NFP * VBLK // 128, 128), lambda j: (j, 0))
                   for _ in range(DIM)],
        out_shape=[jax.ShapeDtypeStruct((NVB * NFP * VBLK // 128, 128),
                                        jnp.float32)
                   for _ in range(DIM)],
    )(embT)


@functools.lru_cache(maxsize=None)
def _sc_gather_build():
    mesh = plsc.VectorSubcoreMesh(core_axis_name="c", subcore_axis_name="s",
                                  num_cores=NC, num_subcores=NS)

    @functools.partial(
        pl.kernel,
        mesh=mesh,
        out_type=jax.ShapeDtypeStruct((NBT * RPT, CS), jnp.float32),
        scratch_types=[
            pltpu.VMEM((NF, B_PER_W), jnp.int32),
            pltpu.VMEM((NF, CS), jnp.int32),
            pltpu.VMEM((NF, CS), jnp.int32),
            pltpu.VMEM((RPT, CS), jnp.float32),
            pltpu.VMEM((NF, CS), jnp.float32),
            pltpu.SemaphoreType.DMA,
            pltpu.SemaphoreType.DMA,
        ],
        compiler_params=pltpu.CompilerParams(use_tc_tiling_on_sc=False,
                                             needs_layout_passes=False),
    )
    def sc_gather(*refs):
        c_hbm = refs[:NF]
        d_tab = refs[NF:NF + DIM]
        fm_hbm = refs[NF + DIM]
        xt_out = refs[NF + DIM + 1]
        (cbuf, idx2, fidx, trbuf, fmv, gsem, fsem) = refs[NF + DIM + 2:]
        wid = lax.axis_index("s") * NC + lax.axis_index("c")
        base_w = wid * B_PER_W

        for f in range(NF):
            pltpu.async_copy(c_hbm[f].at[pl.ds(base_w, B_PER_W)],
                             cbuf.at[f], gsem)
        for f in range(NF):
            pltpu.make_async_copy(c_hbm[f].at[pl.ds(base_w, B_PER_W)],
                                  cbuf.at[f], gsem).wait()

        def chunk_body(c, carry):
            bt = wid * NCHUNK + c

            def igrp(g, carry):
                f = g // (CS // 16)
                k = g % (CS // 16)
                v = cbuf[f, pl.ds(c * CS + k * 16, 16)]
                idx2[f, pl.ds(k * 16, 16)] = (
                    (lax.shift_right_logical(v, VSH) * NFP + f) * VBLK
                    + lax.bitwise_and(v, VBLK - 1))
                fidx[f, pl.ds(k * 16, 16)] = v + VOCAB
                return carry

            lax.fori_loop(0, NF * (CS // 16), igrp, 0)

            def fire(f, carry):
                for d in range(DIM):
                    pltpu.async_copy(d_tab[d].at[idx2.at[f]],
                                     trbuf.at[f * DIM + d], gsem)
                pltpu.async_copy(fm_hbm.at[fidx.at[f]], fmv.at[f], fsem)
                return carry

            lax.fori_loop(0, NF, fire, 0)

            def drain(f, carry):
                for d in range(DIM):
                    pltpu.make_async_copy(d_tab[d].at[idx2.at[f]],
                                          trbuf.at[f * DIM + d], gsem).wait()
                pltpu.make_async_copy(fm_hbm.at[fidx.at[f]],
                                      fmv.at[f], fsem).wait()
                return carry

            lax.fori_loop(0, NF, drain, 0)

            def fsum_grp(k, carry):
                acc = fmv[0, pl.ds(k * 16, 16)]

                def facc(f, a):
                    return a + fmv[f, pl.ds(k * 16, 16)]

                trbuf[NJ, pl.ds(k * 16, 16)] = lax.fori_loop(1, NF, facc, acc)
                return carry

            lax.fori_loop(0, CS // 16, fsum_grp, 0)

            pltpu.sync_copy(trbuf, xt_out.at[pl.ds(bt * RPT, RPT)])
            return carry

        lax.fori_loop(0, NCHUNK, chunk_body, 0)

    return sc_gather


def _tc_body(x_ref, w1_ref, b1_ref, w2_ref, b2_ref, w3_ref, b3_ref,
             wd_ref, bd_ref, o_ref):
    di = lax.broadcasted_iota(jnp.int32, (DIM, NJ), 0)
    ji = lax.broadcasted_iota(jnp.int32, (DIM, NJ), 1)
    ST = (ji % DIM == di).astype(jnp.float32)
    zs = []
    for g in range(8):
        xe = x_ref[pl.ds(g * RPT, NJ), :]
        first = x_ref[pl.ds(g * RPT + NJ, 8), :][0:1, :]
        h = jnp.maximum(
            jnp.dot(w1_ref[...], xe, preferred_element_type=jnp.float32)
            + b1_ref[...], 0.0)
        h = jnp.maximum(
            jnp.dot(w2_ref[...], h, preferred_element_type=jnp.float32)
            + b2_ref[...], 0.0)
        h = jnp.maximum(
            jnp.dot(w3_ref[...], h, preferred_element_type=jnp.float32)
            + b3_ref[...], 0.0)
        deep = jnp.dot(wd_ref[...], h, preferred_element_type=jnp.float32) \
            + bd_ref[...]
        s1 = jnp.dot(ST, xe, preferred_element_type=jnp.float32)
        s2 = jnp.dot(ST, xe * xe, preferred_element_type=jnp.float32)
        second = 0.5 * jnp.sum(s1 * s1 - s2, axis=0, keepdims=True)
        z = first + second + deep
        zs.append(1.0 / (1.0 + jnp.exp(-z)))
    o_ref[...] = jnp.concatenate(zs, axis=0)


def _tc_mlp(XT, W1T, b1, W2T, b2, W3T, b3, WdT, bd):
    grid = (NBT // 8,)
    return pl.pallas_call(
        _tc_body,
        grid=grid,
        in_specs=[
            pl.BlockSpec((8 * RPT, CS), lambda i: (i, 0)),
            pl.BlockSpec(W1T.shape, lambda i: (0, 0)),
            pl.BlockSpec(b1.shape, lambda i: (0, 0)),
            pl.BlockSpec(W2T.shape, lambda i: (0, 0)),
            pl.BlockSpec(b2.shape, lambda i: (0, 0)),
            pl.BlockSpec(W3T.shape, lambda i: (0, 0)),
            pl.BlockSpec(b3.shape, lambda i: (0, 0)),
            pl.BlockSpec(WdT.shape, lambda i: (0, 0)),
            pl.BlockSpec(bd.shape, lambda i: (0, 0)),
        ],
        out_specs=pl.BlockSpec((8, CS), lambda i: (i, 0)),
        out_shape=jax.ShapeDtypeStruct((NBT, CS), jnp.float32),
    )(XT, W1T, b1, W2T, b2, W3T, b3, WdT, bd)


def kernel(C1, C2, C3, C4, C5, C6, C7, C8, C9, C10, C11, C12, C13, C14, C15,
           C16, C17, C18, C19, C20, C21, C22, C23, C24, C25, C26, emb_tables,
           fm_w, W1, b1, W2, b2, W3, b3, Wd, bd):
    fields = [C1, C2, C3, C4, C5, C6, C7, C8, C9, C10, C11, C12, C13, C14,
              C15, C16, C17, C18, C19, C20, C21, C22, C23, C24, C25, C26]
    embT = jnp.transpose(emb_tables, (0, 2, 1))
    d_tabs = [t.reshape(-1) for t in _slice_tables(embT)]
    fm_flat = fm_w.reshape(-1)
    XT = _sc_gather_build()(*fields, *d_tabs, fm_flat)
    out = _tc_mlp(XT, W1.T, b1.reshape(-1, 1), W2.T, b2.reshape(-1, 1),
                  W3.T, b3.reshape(-1, 1), Wd.T, bd.reshape(1, 1))
    return out.reshape(B, 1)
